# Initial kernel scaffold; baseline (speedup 1.0000x reference)
import jax
import jax.numpy as jnp
from jax.experimental import pallas as pl


def kernel(cls_scores, bbox_deltas, anchors):
    raise NotImplementedError("write your pallas kernel here")


# speedup vs baseline: 14.5348x; 14.5348x over previous
import math

import jax
import jax.numpy as jnp
from jax import lax
from jax.experimental import pallas as pl
from jax.experimental.pallas import tpu as pltpu

_N = 20000
_PRE = 1000
_NMS_T = 0.7
_IMG_H = 800.0
_IMG_W = 800.0
_CLIP = math.log(1000.0 / 16.0)
_SUB = 8
_LANE = _N // _SUB


def _stage1(cls_ref, del_ref, anc_ref, p_ref, box_ref):
    w = anc_ref[2] - anc_ref[0]
    h = anc_ref[3] - anc_ref[1]
    cx = anc_ref[0] + 0.5 * w
    cy = anc_ref[1] + 0.5 * h
    nb = cls_ref.shape[0]
    for b in range(nb):
        x0 = cls_ref[b, 0]
        x1 = cls_ref[b, 1]
        m = jnp.maximum(x0, x1)
        e0 = jnp.exp(x0 - m)
        e1 = jnp.exp(x1 - m)
        p_ref[b] = e1 / (e0 + e1)
        dx = del_ref[b, 0]
        dy = del_ref[b, 1]
        dw = jnp.minimum(del_ref[b, 2], _CLIP)
        dh = jnp.minimum(del_ref[b, 3], _CLIP)
        pcx = dx * w + cx
        pcy = dy * h + cy
        pw = jnp.exp(dw) * w
        ph = jnp.exp(dh) * h
        box_ref[b, 0] = jnp.clip(pcx - 0.5 * pw, 0.0, _IMG_W)
        box_ref[b, 1] = jnp.clip(pcy - 0.5 * ph, 0.0, _IMG_H)
        box_ref[b, 2] = jnp.clip(pcx + 0.5 * pw, 0.0, _IMG_W)
        box_ref[b, 3] = jnp.clip(pcy + 0.5 * ph, 0.0, _IMG_H)


def _nms(rows_ref, cols_ref, scol_ref, out_ref, iou_ref):
    x1r = rows_ref[0, 0:1, :]
    y1r = rows_ref[0, 1:2, :]
    x2r = rows_ref[0, 2:3, :]
    y2r = rows_ref[0, 3:4, :]
    x1c = cols_ref[0, :, 0:1]
    y1c = cols_ref[0, :, 1:2]
    x2c = cols_ref[0, :, 2:3]
    y2c = cols_ref[0, :, 3:4]
    ar = (x2r - x1r) * (y2r - y1r)
    ac = (x2c - x1c) * (y2c - y1c)
    wv = jnp.maximum(jnp.minimum(x2c, x2r) - jnp.maximum(x1c, x1r), 0.0)
    hv = jnp.maximum(jnp.minimum(y2c, y2r) - jnp.maximum(y1c, y1r), 0.0)
    inter = wv * hv
    iou_ref[:, :] = inter / (ac + ar - inter + 1e-9)

    iota = lax.broadcasted_iota(jnp.int32, (1, _PRE), 1)

    def body(i, sup):
        onehot = iota == i
        sup_i = jnp.sum(jnp.where(onehot, sup, 0.0))
        keep = jnp.where(sup_i < 0.5, 1.0, 0.0)
        row = iou_ref[pl.ds(i, 1), :]
        hit = jnp.where((row > _NMS_T) & (iota > i), keep, 0.0)
        return jnp.maximum(sup, hit)

    sup = lax.fori_loop(0, _PRE, body, jnp.zeros((1, _PRE), jnp.float32))
    kept = jnp.where(sup < 0.5, 1.0, 0.0)

    ri = lax.broadcasted_iota(jnp.int32, (_PRE, _PRE), 0)
    ci = lax.broadcasted_iota(jnp.int32, (_PRE, _PRE), 1)
    tri = jnp.where(ri <= ci, 1.0, 0.0)
    rank = jnp.dot(kept, tri, preferred_element_type=jnp.float32) - 1.0

    rcol = lax.broadcasted_iota(jnp.int32, (_PRE, 1), 0).astype(jnp.float32)
    perm = jnp.where((rank == rcol) & (kept > 0.5), 1.0, 0.0)
    data = jnp.concatenate([x1c, y1c, x2c, y2c, scol_ref[0]], axis=1)
    out_ref[0] = jnp.dot(perm, data, preferred_element_type=jnp.float32)


def kernel(cls_scores, bbox_deltas, anchors):
    nb = cls_scores.shape[0]
    cls_t = jnp.transpose(cls_scores, (0, 2, 1)).reshape(nb, 2, _SUB, _LANE)
    del_t = jnp.transpose(bbox_deltas, (0, 2, 1)).reshape(nb, 4, _SUB, _LANE)
    anc_t = jnp.transpose(anchors, (1, 0)).reshape(4, _SUB, _LANE)

    p, boxes = pl.pallas_call(
        _stage1,
        out_shape=(
            jax.ShapeDtypeStruct((nb, _SUB, _LANE), jnp.float32),
            jax.ShapeDtypeStruct((nb, 4, _SUB, _LANE), jnp.float32),
        ),
    )(cls_t, del_t, anc_t)

    top_s, idx = lax.top_k(p.reshape(nb, _N), _PRE)
    bsel = jnp.take_along_axis(boxes.reshape(nb, 4, _N), idx[:, None, :], axis=2)
    cols = jnp.transpose(bsel, (0, 2, 1))
    s_col = top_s[:, :, None]

    out = pl.pallas_call(
        _nms,
        grid=(nb,),
        in_specs=[
            pl.BlockSpec((1, 4, _PRE), lambda b: (b, 0, 0)),
            pl.BlockSpec((1, _PRE, 4), lambda b: (b, 0, 0)),
            pl.BlockSpec((1, _PRE, 1), lambda b: (b, 0, 0)),
        ],
        out_specs=pl.BlockSpec((1, _PRE, 5), lambda b: (b, 0, 0)),
        out_shape=jax.ShapeDtypeStruct((nb, _PRE, 5), jnp.float32),
        scratch_shapes=[pltpu.VMEM((_PRE, _PRE), jnp.float32)],
    )(bsel, cols, s_col)
    return out


# speedup vs baseline: 14.6811x; 1.0101x over previous
import math

import jax
import jax.numpy as jnp
from jax import lax
from jax.experimental import pallas as pl
from jax.experimental.pallas import tpu as pltpu

_N = 20000
_PRE = 1000
_NMS_T = 0.7
_IMG_H = 800.0
_IMG_W = 800.0
_CLIP = math.log(1000.0 / 16.0)
_SUB = 8
_LANE = _N // _SUB


def _stage1(cls_ref, del_ref, anc_ref, p_ref, box_ref):
    w = anc_ref[2] - anc_ref[0]
    h = anc_ref[3] - anc_ref[1]
    cx = anc_ref[0] + 0.5 * w
    cy = anc_ref[1] + 0.5 * h
    nb = cls_ref.shape[0]
    for b in range(nb):
        x0 = cls_ref[b, 0]
        x1 = cls_ref[b, 1]
        m = jnp.maximum(x0, x1)
        e0 = jnp.exp(x0 - m)
        e1 = jnp.exp(x1 - m)
        p_ref[b] = e1 / (e0 + e1)
        dx = del_ref[b, 0]
        dy = del_ref[b, 1]
        dw = jnp.minimum(del_ref[b, 2], _CLIP)
        dh = jnp.minimum(del_ref[b, 3], _CLIP)
        pcx = dx * w + cx
        pcy = dy * h + cy
        pw = jnp.exp(dw) * w
        ph = jnp.exp(dh) * h
        box_ref[b, 0] = jnp.clip(pcx - 0.5 * pw, 0.0, _IMG_W)
        box_ref[b, 1] = jnp.clip(pcy - 0.5 * ph, 0.0, _IMG_H)
        box_ref[b, 2] = jnp.clip(pcx + 0.5 * pw, 0.0, _IMG_W)
        box_ref[b, 3] = jnp.clip(pcy + 0.5 * ph, 0.0, _IMG_H)


def _nms(rows_ref, cols_ref, scol_ref, out_ref, iou_ref):
    x1r = rows_ref[0, 0:1, :]
    y1r = rows_ref[0, 1:2, :]
    x2r = rows_ref[0, 2:3, :]
    y2r = rows_ref[0, 3:4, :]
    x1c = cols_ref[0, :, 0:1]
    y1c = cols_ref[0, :, 1:2]
    x2c = cols_ref[0, :, 2:3]
    y2c = cols_ref[0, :, 3:4]
    ar = (x2r - x1r) * (y2r - y1r)
    ac = (x2c - x1c) * (y2c - y1c)
    wv = jnp.maximum(jnp.minimum(x2c, x2r) - jnp.maximum(x1c, x1r), 0.0)
    hv = jnp.maximum(jnp.minimum(y2c, y2r) - jnp.maximum(y1c, y1r), 0.0)
    inter = wv * hv
    iou_ref[:, :] = inter / (ac + ar - inter + 1e-9)

    iota = lax.broadcasted_iota(jnp.int32, (1, _PRE), 1)

    def body(i, sup):
        onehot = iota == i
        sup_i = jnp.sum(jnp.where(onehot, sup, 0.0))
        keep = jnp.where(sup_i < 0.5, 1.0, 0.0)
        row = iou_ref[pl.ds(i, 1), :]
        hit = jnp.where((row > _NMS_T) & (iota > i), keep, 0.0)
        return jnp.maximum(sup, hit)

    sup = lax.fori_loop(0, _PRE, body, jnp.zeros((1, _PRE), jnp.float32),
                        unroll=8)
    kept = jnp.where(sup < 0.5, 1.0, 0.0)

    ri = lax.broadcasted_iota(jnp.int32, (_PRE, _PRE), 0)
    ci = lax.broadcasted_iota(jnp.int32, (_PRE, _PRE), 1)
    tri = jnp.where(ri <= ci, 1.0, 0.0)
    rank = jnp.dot(kept, tri, preferred_element_type=jnp.float32) - 1.0

    rcol = lax.broadcasted_iota(jnp.int32, (_PRE, 1), 0).astype(jnp.float32)
    perm = jnp.where((rank == rcol) & (kept > 0.5), 1.0, 0.0)
    data = jnp.concatenate([x1c, y1c, x2c, y2c, scol_ref[0]], axis=1)
    out_ref[0] = jnp.dot(perm, data, preferred_element_type=jnp.float32)


def kernel(cls_scores, bbox_deltas, anchors):
    nb = cls_scores.shape[0]
    cls_t = jnp.transpose(cls_scores, (0, 2, 1)).reshape(nb, 2, _SUB, _LANE)
    del_t = jnp.transpose(bbox_deltas, (0, 2, 1)).reshape(nb, 4, _SUB, _LANE)
    anc_t = jnp.transpose(anchors, (1, 0)).reshape(4, _SUB, _LANE)

    p, boxes = pl.pallas_call(
        _stage1,
        out_shape=(
            jax.ShapeDtypeStruct((nb, _SUB, _LANE), jnp.float32),
            jax.ShapeDtypeStruct((nb, 4, _SUB, _LANE), jnp.float32),
        ),
    )(cls_t, del_t, anc_t)

    top_s, idx = lax.top_k(p.reshape(nb, _N), _PRE)
    bsel = jnp.take_along_axis(boxes.reshape(nb, 4, _N), idx[:, None, :], axis=2)
    cols = jnp.transpose(bsel, (0, 2, 1))
    s_col = top_s[:, :, None]

    out = pl.pallas_call(
        _nms,
        grid=(nb,),
        in_specs=[
            pl.BlockSpec((1, 4, _PRE), lambda b: (b, 0, 0)),
            pl.BlockSpec((1, _PRE, 4), lambda b: (b, 0, 0)),
            pl.BlockSpec((1, _PRE, 1), lambda b: (b, 0, 0)),
        ],
        out_specs=pl.BlockSpec((1, _PRE, 5), lambda b: (b, 0, 0)),
        out_shape=jax.ShapeDtypeStruct((nb, _PRE, 5), jnp.float32),
        scratch_shapes=[pltpu.VMEM((_PRE, _PRE), jnp.float32)],
    )(bsel, cols, s_col)
    return out
